# baseline (speedup 1.0000x reference)
import functools

import jax
import jax.numpy as jnp
from jax import lax
from jax.experimental import pallas as pl
from jax.experimental.pallas import tpu as pltpu
from jax.experimental.pallas import tpu_sc as plsc

_K = 2
_E = 8
_D = 1024
_T = 4096
_B = 256
_P = _T * _K + _E * _B
_NB = _P // _B
_TT = 512

_NW = 32
_TPW = _T // _NW
_CH = 32
_NCH = _TPW // _CH

@functools.cache
def _get_mesh():
    return plsc.VectorSubcoreMesh(core_axis_name="c", subcore_axis_name="s")


def _router_body(x_ref, wg_ref, bg_ref,
                 prob_ref, pb0_ref, pb1_ref, pos0_ref, pos1_ref, beid_ref,
                 carry_s, idx_s, rank_s, probs_s, starts_s, ends_s):
    g = pl.program_id(0)

    @pl.when(g == 0)
    def _():
        carry_s[...] = jnp.zeros_like(carry_s)

    @pl.when(g < 8)
    def _():
        ds = pl.ds(g * _TT, _TT)
        scores = jnp.dot(x_ref[...], wg_ref[...],
                         preferred_element_type=jnp.float32) + bg_ref[...]
        col = lax.broadcasted_iota(jnp.int32, scores.shape, 1)
        s1 = jnp.max(scores, axis=1, keepdims=True)
        i1 = jnp.min(jnp.where(scores == s1, col, _E), axis=1, keepdims=True)
        masked = jnp.where(col == i1, -jnp.inf, scores)
        s2 = jnp.max(masked, axis=1, keepdims=True)
        i2 = jnp.min(jnp.where(masked == s2, col, _E), axis=1, keepdims=True)
        e2 = jnp.exp(s2 - s1)
        denom = 1.0 + e2
        idx_s[ds, 0:1] = i1
        idx_s[ds, 1:2] = i2
        probs_s[ds, 0:1] = 1.0 / denom
        probs_s[ds, 1:2] = e2 / denom
        oh0 = (col == i1).astype(jnp.float32)
        oh1 = (col == i2).astype(jnp.float32)
        rowsum = oh0 + oh1
        r_io = lax.broadcasted_iota(jnp.int32, (_TT, _TT), 0)
        c_io = lax.broadcasted_iota(jnp.int32, (_TT, _TT), 1)
        ltri = (r_io > c_io).astype(jnp.float32)
        prefix = jnp.dot(ltri, rowsum,
                         preferred_element_type=jnp.float32) + carry_s[...]
        rank_s[ds, 0:1] = jnp.sum(prefix * oh0, axis=1, keepdims=True)
        rank_s[ds, 1:2] = jnp.sum(prefix * oh1, axis=1, keepdims=True)
        carry_s[...] = carry_s[...] + jnp.sum(rowsum, axis=0, keepdims=True)

    @pl.when(g == 8)
    def _():
        counts = carry_s[...]
        padded = jnp.floor((counts + (_B - 1)) * (1.0 / _B)) * _B
        r8 = lax.broadcasted_iota(jnp.int32, (_E, _E), 0)
        c8 = lax.broadcasted_iota(jnp.int32, (_E, _E), 1)
        ustrict = (r8 < c8).astype(jnp.float32)
        starts = jnp.dot(padded, ustrict, preferred_element_type=jnp.float32)
        starts_s[...] = starts
        ends_s[...] = starts + padded

    @pl.when(g >= 8)
    def _():
        ds = pl.ds((g - 8) * _TT, _TT)
        starts = starts_s[...]
        i1 = idx_s[ds, 0:1]
        i2 = idx_s[ds, 1:2]
        col = lax.broadcasted_iota(jnp.int32, (_TT, _E), 1)
        sel0 = jnp.sum(jnp.where(col == i1, starts, 0.0), axis=1,
                       keepdims=True)
        sel1 = jnp.sum(jnp.where(col == i2, starts, 0.0), axis=1,
                       keepdims=True)
        pos0_ref[...] = (sel0 + rank_s[ds, 0:1]).astype(jnp.int32)
        pos1_ref[...] = (sel1 + rank_s[ds, 1:2]).astype(jnp.int32)
        p = probs_s[ds, :]
        prob_ref[...] = p
        pb0_ref[...] = jnp.broadcast_to(p[:, 0:1], (_TT, _L))
        pb1_ref[...] = jnp.broadcast_to(p[:, 1:2], (_TT, _L))
        bvals = lax.broadcasted_iota(jnp.int32, (_NB, _E), 0
                                     ).astype(jnp.float32) * float(_B)
        cmp = (bvals >= ends_s[...]).astype(jnp.float32)
        beid_ref[...] = jnp.minimum(
            jnp.sum(cmp, axis=1, keepdims=True), float(_E - 1)
        ).astype(jnp.int32)


def _router(x, wg, bg2):
    def _xmap(g):
        return (jnp.where(g < 8, g, 7), 0)

    def _omap(g):
        return (g % 8, 0)

    return pl.pallas_call(
        _router_body,
        grid=(16,),
        in_specs=[
            pl.BlockSpec((_TT, _D), _xmap),
            pl.BlockSpec((_D, _E), lambda g: (0, 0)),
            pl.BlockSpec((1, _E), lambda g: (0, 0)),
        ],
        out_specs=[
            pl.BlockSpec((_TT, _K), _omap),
            pl.BlockSpec((_TT, _L), _omap),
            pl.BlockSpec((_TT, _L), _omap),
            pl.BlockSpec((_TT, 1), _omap),
            pl.BlockSpec((_TT, 1), _omap),
            pl.BlockSpec((_NB, 1), lambda g: (0, 0)),
        ],
        out_shape=[
            jax.ShapeDtypeStruct((_T, _K), jnp.float32),
            jax.ShapeDtypeStruct((_T, _L), jnp.float32),
            jax.ShapeDtypeStruct((_T, _L), jnp.float32),
            jax.ShapeDtypeStruct((_T, 1), jnp.int32),
            jax.ShapeDtypeStruct((_T, 1), jnp.int32),
            jax.ShapeDtypeStruct((_NB, 1), jnp.int32),
        ],
        scratch_shapes=[
            pltpu.VMEM((1, _E), jnp.float32),
            pltpu.VMEM((_T, _K), jnp.int32),
            pltpu.VMEM((_T, _K), jnp.float32),
            pltpu.VMEM((_T, _K), jnp.float32),
            pltpu.VMEM((1, _E), jnp.float32),
            pltpu.VMEM((1, _E), jnp.float32),
        ],
    )(x, wg, bg2)


def _sc_dispatch(x, pos0, pos1):
    @functools.partial(
        pl.kernel, mesh=_get_mesh(),
        out_type=jax.ShapeDtypeStruct((_P, _D), jnp.float32),
        scratch_types=[
            pltpu.VMEM((_CH, _D), jnp.float32),
            pltpu.VMEM((_CH,), jnp.int32),
            pltpu.VMEM((_CH,), jnp.int32),
            pltpu.SemaphoreType.DMA,
        ],
    )
    def k(x_hbm, p0_hbm, p1_hbm, xs_hbm, rows_v, i0_v, i1_v, sem):
        wid = lax.axis_index("s") * 2 + lax.axis_index("c")
        base = wid * _TPW
        for c in range(_NCH):
            off = base + c * _CH
            pltpu.sync_copy(p0_hbm.at[pl.ds(off, _CH)], i0_v)
            pltpu.sync_copy(p1_hbm.at[pl.ds(off, _CH)], i1_v)
            pltpu.sync_copy(x_hbm.at[pl.ds(off, _CH)], rows_v)
            cp0 = pltpu.async_copy(rows_v, xs_hbm.at[i0_v], sem)
            cp1 = pltpu.async_copy(rows_v, xs_hbm.at[i1_v], sem)
            cp0.wait()
            cp1.wait()

    return k(x, pos0, pos1)


def _gmm_body(eid_ref, xs_ref, we_ref, be_ref, ys_ref):
    ys_ref[...] = jnp.dot(xs_ref[...], we_ref[0],
                          preferred_element_type=jnp.float32) + be_ref[0]


def _grouped_matmul(block_eid, xs, we, be):
    grid_spec = pltpu.PrefetchScalarGridSpec(
        num_scalar_prefetch=1,
        grid=(_NB,),
        in_specs=[
            pl.BlockSpec((_B, _D), lambda b, eid: (b, 0)),
            pl.BlockSpec((1, _D, _D), lambda b, eid: (eid[b], 0, 0)),
            pl.BlockSpec((1, 1, _D), lambda b, eid: (eid[b], 0, 0)),
        ],
        out_specs=pl.BlockSpec((_B, _D), lambda b, eid: (b, 0)),
    )
    return pl.pallas_call(
        _gmm_body,
        grid_spec=grid_spec,
        out_shape=jax.ShapeDtypeStruct((_P, _D), jnp.float32),
    )(block_eid, xs, we, be)


_CC = 16
_NCC = _TPW // _CC
_L = 16


def _sc_collect_combine(ys, pos0, pos1, pb0, pb1):
    @functools.partial(
        pl.kernel, mesh=_get_mesh(),
        out_type=jax.ShapeDtypeStruct((_T, _D), jnp.float32),
        scratch_types=[
            pltpu.VMEM((_CC, _D), jnp.float32),
            pltpu.VMEM((_CC, _D), jnp.float32),
            pltpu.VMEM((_CC, _D), jnp.float32),
            pltpu.VMEM((_CC, _D), jnp.float32),
            pltpu.VMEM((_CC, _D), jnp.float32),
            pltpu.VMEM((_NCC, _CC), jnp.int32),
            pltpu.VMEM((_NCC, _CC), jnp.int32),
            pltpu.VMEM((_TPW, _L), jnp.float32),
            pltpu.VMEM((_TPW, _L), jnp.float32),
            pltpu.SemaphoreType.DMA,
            pltpu.SemaphoreType.DMA,
        ],
    )
    def k(ys_hbm, p0_hbm, p1_hbm, pb0_hbm, pb1_hbm, out_hbm,
          a0_v, a1_v, b0_v, b1_v, o_v, i0_v, i1_v, q0_v, q1_v, s0, s1):
        wid = lax.axis_index("s") * 2 + lax.axis_index("c")
        base = wid * _TPW
        a_bufs, b_bufs, sems = (a0_v, a1_v), (b0_v, b1_v), (s0, s1)
        pltpu.sync_copy(pb0_hbm.at[pl.ds(base, _TPW)], q0_v)
        pltpu.sync_copy(pb1_hbm.at[pl.ds(base, _TPW)], q1_v)
        for c in range(_NCC):
            pltpu.sync_copy(p0_hbm.at[pl.ds(base + c * _CC, _CC)],
                            i0_v.at[c])
            pltpu.sync_copy(p1_hbm.at[pl.ds(base + c * _CC, _CC)],
                            i1_v.at[c])

        def issue(c):
            s = sems[c % 2]
            ca = pltpu.async_copy(ys_hbm.at[i0_v.at[c]], a_bufs[c % 2], s)
            cb = pltpu.async_copy(ys_hbm.at[i1_v.at[c]], b_bufs[c % 2], s)
            return ca, cb

        pend = issue(0)
        for c in range(_NCC):
            nxt = issue(c + 1) if c + 1 < _NCC else None
            pend[0].wait()
            pend[1].wait()
            a_v, b_v = a_bufs[c % 2], b_bufs[c % 2]

            def tok_body(t, carry):
                p0s = q0_v[c * _CC + t, :]
                p1s = q1_v[c * _CC + t, :]
                for j in range(_D // _L):
                    sl = pl.ds(j * _L, _L)
                    o_v[t, sl] = a_v[t, sl] * p0s + b_v[t, sl] * p1s
                return carry

            lax.fori_loop(0, _CC, tok_body, 0)
            pltpu.sync_copy(o_v, out_hbm.at[pl.ds(base + c * _CC, _CC)])
            pend = nxt

    return k(ys, pos0, pos1, pb0, pb1)


def kernel(inputs, Wg, bg, We, be):
    probs, pb0, pb1, pos0, pos1, beid = _router(inputs, Wg,
                                                bg.reshape(1, _E))
    pos0, pos1 = pos0.reshape(_T), pos1.reshape(_T)
    xs = _sc_dispatch(inputs, pos0, pos1)
    ys = _grouped_matmul(beid.reshape(_NB), xs, We, be.reshape(_E, 1, _D))
    out = _sc_collect_combine(ys, pos0, pos1, pb0, pb1)
    return (out, probs)


# speedup vs baseline: 1.0279x; 1.0279x over previous
import functools

import jax
import jax.numpy as jnp
from jax import lax
from jax.experimental import pallas as pl
from jax.experimental.pallas import tpu as pltpu
from jax.experimental.pallas import tpu_sc as plsc

_K = 2
_E = 8
_D = 1024
_T = 4096
_B = 256
_P = _T * _K + _E * _B
_NB = _P // _B
_TT = 512

_NW = 32
_TPW = _T // _NW
_CH = 32
_NCH = _TPW // _CH

@functools.cache
def _get_mesh():
    return plsc.VectorSubcoreMesh(core_axis_name="c", subcore_axis_name="s")


def _router_body(x_ref, wg_ref, bg_ref, prob_ref, pb0_ref, pb1_ref,
                 idx0_ref, idx1_ref, rank0_ref, rank1_ref, cnt_ref,
                 carry_s):
    g = pl.program_id(0)

    @pl.when(g == 0)
    def _():
        carry_s[...] = jnp.zeros_like(carry_s)

    scores = jnp.dot(x_ref[...], wg_ref[...],
                     preferred_element_type=jnp.float32) + bg_ref[...]
    col = lax.broadcasted_iota(jnp.int32, scores.shape, 1)
    s1 = jnp.max(scores, axis=1, keepdims=True)
    i1 = jnp.min(jnp.where(scores == s1, col, _E), axis=1, keepdims=True)
    masked = jnp.where(col == i1, -jnp.inf, scores)
    s2 = jnp.max(masked, axis=1, keepdims=True)
    i2 = jnp.min(jnp.where(masked == s2, col, _E), axis=1, keepdims=True)
    e2 = jnp.exp(s2 - s1)
    denom = 1.0 + e2
    p0 = 1.0 / denom
    p1 = e2 / denom
    idx0_ref[...] = i1
    idx1_ref[...] = i2
    prob_ref[:, 0:1] = p0
    prob_ref[:, 1:2] = p1
    pb0_ref[...] = jnp.broadcast_to(p0, (_TT, _L))
    pb1_ref[...] = jnp.broadcast_to(p1, (_TT, _L))
    oh0 = (col == i1).astype(jnp.float32)
    oh1 = (col == i2).astype(jnp.float32)
    rowsum = oh0 + oh1
    r_io = lax.broadcasted_iota(jnp.int32, (_TT, _TT), 0)
    c_io = lax.broadcasted_iota(jnp.int32, (_TT, _TT), 1)
    ltri = (r_io > c_io).astype(jnp.float32)
    prefix = jnp.dot(ltri, rowsum,
                     preferred_element_type=jnp.float32) + carry_s[...]
    rank0_ref[...] = jnp.sum(prefix * oh0, axis=1,
                             keepdims=True).astype(jnp.int32)
    rank1_ref[...] = jnp.sum(prefix * oh1, axis=1,
                             keepdims=True).astype(jnp.int32)
    carry_new = carry_s[...] + jnp.sum(rowsum, axis=0, keepdims=True)
    carry_s[...] = carry_new
    cnt_ref[...] = carry_new.astype(jnp.int32)


def _router(x, wg, bg2):
    def _omap(g):
        return (g, 0)

    return pl.pallas_call(
        _router_body,
        grid=(_T // _TT,),
        in_specs=[
            pl.BlockSpec((_TT, _D), lambda g: (g, 0)),
            pl.BlockSpec((_D, _E), lambda g: (0, 0)),
            pl.BlockSpec((1, _E), lambda g: (0, 0)),
        ],
        out_specs=[
            pl.BlockSpec((_TT, _K), _omap),
            pl.BlockSpec((_TT, _L), _omap),
            pl.BlockSpec((_TT, _L), _omap),
            pl.BlockSpec((_TT, 1), _omap),
            pl.BlockSpec((_TT, 1), _omap),
            pl.BlockSpec((_TT, 1), _omap),
            pl.BlockSpec((_TT, 1), _omap),
            pl.BlockSpec((1, _E), lambda g: (0, 0)),
        ],
        out_shape=[
            jax.ShapeDtypeStruct((_T, _K), jnp.float32),
            jax.ShapeDtypeStruct((_T, _L), jnp.float32),
            jax.ShapeDtypeStruct((_T, _L), jnp.float32),
            jax.ShapeDtypeStruct((_T, 1), jnp.int32),
            jax.ShapeDtypeStruct((_T, 1), jnp.int32),
            jax.ShapeDtypeStruct((_T, 1), jnp.int32),
            jax.ShapeDtypeStruct((_T, 1), jnp.int32),
            jax.ShapeDtypeStruct((1, _E), jnp.int32),
        ],
        scratch_shapes=[pltpu.VMEM((1, _E), jnp.float32)],
    )(x, wg, bg2)


_RC = 32
_NRC = _TPW // _RC


def _dispatch_plan(idx0, idx1, rank0, rank1, cnt):
    counts = cnt.reshape(_E)
    padded = ((counts + _B - 1) // _B) * _B
    starts = jnp.concatenate(
        [jnp.zeros((1,), padded.dtype), jnp.cumsum(padded)[:-1]])
    ends = starts + padded
    cols = jnp.arange(_E, dtype=jnp.int32)[None, :]
    sel0 = jnp.sum(jnp.where(idx0.reshape(_T, 1) == cols, starts[None, :],
                             0), axis=1)
    sel1 = jnp.sum(jnp.where(idx1.reshape(_T, 1) == cols, starts[None, :],
                             0), axis=1)
    pos0 = (sel0 + rank0.reshape(_T)).astype(jnp.int32)
    pos1 = (sel1 + rank1.reshape(_T)).astype(jnp.int32)
    beid = jnp.minimum(
        jnp.sum((jnp.arange(_NB, dtype=jnp.int32)[:, None] * _B
                 >= ends[None, :]).astype(jnp.int32), axis=1),
        _E - 1).astype(jnp.int32)
    return pos0, pos1, beid


def _sc_dispatch(x, pos0, pos1):
    @functools.partial(
        pl.kernel, mesh=_get_mesh(),
        out_type=jax.ShapeDtypeStruct((_P, _D), jnp.float32),
        scratch_types=[
            pltpu.VMEM((_RC, _D), jnp.float32),
            pltpu.VMEM((_RC, _D), jnp.float32),
            pltpu.VMEM((_NRC, _RC), jnp.int32),
            pltpu.VMEM((_NRC, _RC), jnp.int32),
            pltpu.SemaphoreType.DMA,
            pltpu.SemaphoreType.DMA,
        ],
    )
    def k(x_hbm, p0_hbm, p1_hbm, xs_hbm,
          rows0_v, rows1_v, p0_v, p1_v, s0, s1):
        wid = lax.axis_index("s") * 2 + lax.axis_index("c")
        base = wid * _TPW
        for r in range(_NRC):
            pltpu.sync_copy(p0_hbm.at[pl.ds(base + r * _RC, _RC)],
                            p0_v.at[r])
            pltpu.sync_copy(p1_hbm.at[pl.ds(base + r * _RC, _RC)],
                            p1_v.at[r])
        rows, sems = (rows0_v, rows1_v), (s0, s1)
        pend = [None, None]
        for c in range(_NRC):
            b = c % 2
            if pend[b] is not None:
                pend[b][0].wait()
                pend[b][1].wait()
            pltpu.sync_copy(x_hbm.at[pl.ds(base + c * _RC, _RC)], rows[b])
            cp0 = pltpu.async_copy(rows[b], xs_hbm.at[p0_v.at[c]], sems[b])
            cp1 = pltpu.async_copy(rows[b], xs_hbm.at[p1_v.at[c]], sems[b])
            pend[b] = (cp0, cp1)
        for b in range(2):
            if pend[b] is not None:
                pend[b][0].wait()
                pend[b][1].wait()

    return k(x, pos0, pos1)


def _gmm_body(eid_ref, xs_ref, we_ref, be_ref, ys_ref):
    ys_ref[...] = jnp.dot(xs_ref[...], we_ref[0],
                          preferred_element_type=jnp.float32) + be_ref[0]


def _grouped_matmul(block_eid, xs, we, be):
    grid_spec = pltpu.PrefetchScalarGridSpec(
        num_scalar_prefetch=1,
        grid=(_NB,),
        in_specs=[
            pl.BlockSpec((_B, _D), lambda b, eid: (b, 0)),
            pl.BlockSpec((1, _D, _D), lambda b, eid: (eid[b], 0, 0)),
            pl.BlockSpec((1, 1, _D), lambda b, eid: (eid[b], 0, 0)),
        ],
        out_specs=pl.BlockSpec((_B, _D), lambda b, eid: (b, 0)),
    )
    return pl.pallas_call(
        _gmm_body,
        grid_spec=grid_spec,
        out_shape=jax.ShapeDtypeStruct((_P, _D), jnp.float32),
    )(block_eid, xs, we, be)


_CC = 16
_NCC = _TPW // _CC
_L = 16


def _sc_collect_combine(ys, pos0, pos1, pb0, pb1):
    @functools.partial(
        pl.kernel, mesh=_get_mesh(),
        out_type=jax.ShapeDtypeStruct((_T, _D), jnp.float32),
        scratch_types=[
            pltpu.VMEM((_CC, _D), jnp.float32),
            pltpu.VMEM((_CC, _D), jnp.float32),
            pltpu.VMEM((_CC, _D), jnp.float32),
            pltpu.VMEM((_CC, _D), jnp.float32),
            pltpu.VMEM((_CC, _D), jnp.float32),
            pltpu.VMEM((_NCC, _CC), jnp.int32),
            pltpu.VMEM((_NCC, _CC), jnp.int32),
            pltpu.VMEM((_TPW, _L), jnp.float32),
            pltpu.VMEM((_TPW, _L), jnp.float32),
            pltpu.SemaphoreType.DMA,
            pltpu.SemaphoreType.DMA,
        ],
    )
    def k(ys_hbm, p0_hbm, p1_hbm, pb0_hbm, pb1_hbm, out_hbm,
          a0_v, a1_v, b0_v, b1_v, o_v, i0_v, i1_v, q0_v, q1_v, s0, s1):
        wid = lax.axis_index("s") * 2 + lax.axis_index("c")
        base = wid * _TPW
        a_bufs, b_bufs, sems = (a0_v, a1_v), (b0_v, b1_v), (s0, s1)
        pltpu.sync_copy(pb0_hbm.at[pl.ds(base, _TPW)], q0_v)
        pltpu.sync_copy(pb1_hbm.at[pl.ds(base, _TPW)], q1_v)
        for c in range(_NCC):
            pltpu.sync_copy(p0_hbm.at[pl.ds(base + c * _CC, _CC)],
                            i0_v.at[c])
            pltpu.sync_copy(p1_hbm.at[pl.ds(base + c * _CC, _CC)],
                            i1_v.at[c])

        def issue(c):
            s = sems[c % 2]
            ca = pltpu.async_copy(ys_hbm.at[i0_v.at[c]], a_bufs[c % 2], s)
            cb = pltpu.async_copy(ys_hbm.at[i1_v.at[c]], b_bufs[c % 2], s)
            return ca, cb

        pend = issue(0)
        for c in range(_NCC):
            nxt = issue(c + 1) if c + 1 < _NCC else None
            pend[0].wait()
            pend[1].wait()
            a_v, b_v = a_bufs[c % 2], b_bufs[c % 2]

            def tok_body(t, carry):
                p0s = q0_v[c * _CC + t, :]
                p1s = q1_v[c * _CC + t, :]
                for j in range(_D // _L):
                    sl = pl.ds(j * _L, _L)
                    o_v[t, sl] = a_v[t, sl] * p0s + b_v[t, sl] * p1s
                return carry

            lax.fori_loop(0, _CC, tok_body, 0)
            pltpu.sync_copy(o_v, out_hbm.at[pl.ds(base + c * _CC, _CC)])
            pend = nxt

    return k(ys, pos0, pos1, pb0, pb1)


def kernel(inputs, Wg, bg, We, be):
    probs, pb0, pb1, idx0, idx1, rank0, rank1, cnt = _router(
        inputs, Wg, bg.reshape(1, _E))
    pos0, pos1, beid = _dispatch_plan(idx0, idx1, rank0, rank1, cnt)
    xs = _sc_dispatch(inputs, pos0, pos1)
    ys = _grouped_matmul(beid, xs, We, be.reshape(_E, 1, _D))
    out = _sc_collect_combine(ys, pos0, pos1, pb0, pb1)
    return (out, probs)
